# baseline (speedup 1.0000x reference)
import functools

import jax
import jax.numpy as jnp
from jax import lax
from jax.experimental import pallas as pl
from jax.experimental.pallas import tpu as pltpu
from jax.experimental.pallas import tpu_sc as plsc

D = 32


@functools.lru_cache(maxsize=None)
def _build(B: int, T: int, V: int):
    info = plsc.get_sparse_core_info()
    NC, NS = info.num_cores, info.num_subcores
    NW = NC * NS
    assert B % NW == 0
    b_per_w = B // NW
    f_per_w = b_per_w * T
    CH_B = 16
    CHF = CH_B * T
    n_ch = b_per_w // CH_B
    assert n_ch % 2 == 0

    mesh = plsc.VectorSubcoreMesh(core_axis_name="c", subcore_axis_name="s")

    @functools.partial(
        pl.kernel,
        mesh=mesh,
        out_type=jax.ShapeDtypeStruct((B, T, D), jnp.float32),
        compiler_params=pltpu.CompilerParams(use_tc_tiling_on_sc=False),
        scratch_types=[
            pltpu.VMEM((f_per_w,), jnp.int32),
            pltpu.VMEM((2, CHF, D), jnp.float32),
            pltpu.SemaphoreType.DMA,
            pltpu.SemaphoreType.DMA,
        ],
    )
    def gather_kernel(idx_hbm, table_hbm, out_hbm, idx_v, rows_v, gsem, osem):
        wid = lax.axis_index("s") * NC + lax.axis_index("c")
        bbase = wid * b_per_w
        pltpu.sync_copy(idx_hbm.at[pl.ds(wid * f_per_w, f_per_w)], idx_v)

        def gather(g, b):
            pltpu.async_copy(
                table_hbm.at[idx_v.at[pl.ds(g * CHF, CHF)]], rows_v.at[b],
                gsem)

        def wait_gather(g, b):
            pltpu.make_async_copy(
                table_hbm.at[idx_v.at[pl.ds(g * CHF, CHF)]], rows_v.at[b],
                gsem).wait()

        def write(g, b):
            for k in range(CH_B):
                pltpu.async_copy(
                    rows_v.at[b].at[pl.ds(k * T, T)],
                    out_hbm.at[bbase + g * CH_B + k], osem)

        def drain_writes():
            for _ in range(CH_B):
                pltpu.make_async_copy(
                    rows_v.at[0].at[pl.ds(0, T)], out_hbm.at[0], osem).wait()

        gather(0, 0)

        def super_step(p, _):
            for b in range(2):
                g = p * 2 + b

                @pl.when(g >= 1)
                def _():
                    drain_writes()

                @pl.when(g + 1 < n_ch)
                def _():
                    gather(g + 1, 1 - b)

                wait_gather(g, b)
                write(g, b)
            return 0

        lax.fori_loop(0, n_ch // 2, super_step, 0)
        drain_writes()

    return gather_kernel


def kernel(input_ids, W):
    Bt, T = input_ids.shape
    flat = input_ids.reshape(Bt * T).astype(jnp.int32)
    fn = _build(Bt, T, W.shape[0])
    return fn(flat, W)
